# baseline (speedup 1.0000x reference)
import functools

import jax
import jax.numpy as jnp
from jax import lax
from jax.experimental import pallas as pl
from jax.experimental.pallas import tpu as pltpu
from jax.experimental.pallas import tpu_sc as plsc

H = 768
L = 16
HC = H // L
NC, NS = 2, 16
NW = NC * NS
EPS = 1e-5


def _rsqrt(x):
    i = plsc.bitcast(x, jnp.int32)
    i = jnp.int32(0x5F3759DF) - lax.shift_right_arithmetic(i, 1)
    y = plsc.bitcast(i, jnp.float32)
    for _ in range(3):
        y = y * (1.5 - 0.5 * x * y * y)
    return y


def _allsum(x):
    return jnp.broadcast_to(plsc.cumsum(x)[L - 1], (L,))


def _ln_rows(rows_v, w_v, n_rows):
    inv_h = jnp.float32(1.0 / H)

    @pl.loop(0, n_rows)
    def _row(r):
        s = jnp.zeros((L,), jnp.float32)
        q = jnp.zeros((L,), jnp.float32)
        for j in range(HC):
            v = rows_v[r, pl.ds(j * L, L)]
            s = s + v
            q = q + v * v
        mean = _allsum(s) * inv_h
        var = _allsum(q) * inv_h - mean * mean
        rstd = _rsqrt(var + EPS)
        shift = -mean * rstd
        for j in range(HC):
            v = rows_v[r, pl.ds(j * L, L)]
            w = w_v[pl.ds(j * L, L)]
            rows_v[r, pl.ds(j * L, L)] = (v * rstd + shift) * w


def _make_sc_kernel(n_tokens):
    tw = n_tokens // NW
    chunk = 64
    n_chunks = tw // chunk

    mesh = plsc.VectorSubcoreMesh(
        core_axis_name="c", subcore_axis_name="s",
        num_cores=NC, num_subcores=NS)

    @functools.partial(
        pl.kernel,
        out_type=jax.ShapeDtypeStruct((n_tokens, H), jnp.float32),
        mesh=mesh,
        scratch_types=[
            pltpu.VMEM((tw,), jnp.int32),
            pltpu.VMEM((chunk, H), jnp.float32),
            pltpu.VMEM((H,), jnp.float32),
            pltpu.SemaphoreType.DMA,
        ],
        compiler_params=pltpu.CompilerParams(needs_layout_passes=False),
    )
    def sc_kernel(ids_hbm, table_hbm, w_hbm, out_hbm, idx_v, rows_v, w_v, sem):
        wid = lax.axis_index("s") * NC + lax.axis_index("c")
        base = wid * tw
        pltpu.sync_copy(ids_hbm.at[pl.ds(base, tw)], idx_v)
        pltpu.sync_copy(w_hbm, w_v)

        @pl.loop(0, n_chunks)
        def _chunk(g):
            row0 = g * chunk
            pltpu.async_copy(
                table_hbm.at[idx_v.at[pl.ds(row0, chunk)]], rows_v, sem
            ).wait()
            _ln_rows(rows_v, w_v, chunk)
            pltpu.sync_copy(rows_v, out_hbm.at[pl.ds(base + row0, chunk)])

    return sc_kernel


def kernel(input_ids, tok_table, ln_weight):
    b, s = input_ids.shape
    ids = input_ids.reshape(b * s)
    out = _make_sc_kernel(b * s)(ids, tok_table, ln_weight)
    return out.reshape(b, s, H)


# speedup vs baseline: 1.6985x; 1.6985x over previous
import functools

import jax
import jax.numpy as jnp
from jax import lax
from jax.experimental import pallas as pl
from jax.experimental.pallas import tpu as pltpu
from jax.experimental.pallas import tpu_sc as plsc

H = 768
L = 16
HC = H // L
NC, NS = 2, 16
NW = NC * NS
EPS = 1e-5


def _rsqrt(x):
    i = plsc.bitcast(x, jnp.int32)
    i = jnp.int32(0x5F3759DF) - lax.shift_right_arithmetic(i, 1)
    y = plsc.bitcast(i, jnp.float32)
    for _ in range(3):
        y = y * (1.5 - 0.5 * x * y * y)
    return y


def _allsum(x):
    return jnp.broadcast_to(plsc.cumsum(x)[L - 1], (L,))


def _ln_rows(rows_v, w_v, n_rows):
    inv_h = jnp.float32(1.0 / H)

    @plsc.parallel_loop(0, n_rows)
    def _row(r):
        vs = []
        s = [jnp.zeros((L,), jnp.float32) for _ in range(4)]
        q = [jnp.zeros((L,), jnp.float32) for _ in range(4)]
        for j in range(HC):
            v = rows_v[r, pl.ds(j * L, L)]
            vs.append(v)
            s[j % 4] = s[j % 4] + v
            q[j % 4] = q[j % 4] + v * v
        stot = (s[0] + s[1]) + (s[2] + s[3])
        qtot = (q[0] + q[1]) + (q[2] + q[3])
        mean = _allsum(stot) * inv_h
        var = _allsum(qtot) * inv_h - mean * mean
        rstd = _rsqrt(var + EPS)
        shift = -mean * rstd
        for j in range(HC):
            w = w_v[pl.ds(j * L, L)]
            rows_v[r, pl.ds(j * L, L)] = (vs[j] * rstd + shift) * w


def _make_sc_kernel(n_tokens):
    tw = n_tokens // NW
    chunk = 64
    n_chunks = tw // chunk

    mesh = plsc.VectorSubcoreMesh(
        core_axis_name="c", subcore_axis_name="s",
        num_cores=NC, num_subcores=NS)

    @functools.partial(
        pl.kernel,
        out_type=jax.ShapeDtypeStruct((n_tokens, H), jnp.float32),
        mesh=mesh,
        scratch_types=[
            pltpu.VMEM((tw,), jnp.int32),
            pltpu.VMEM((chunk, H), jnp.float32),
            pltpu.VMEM((chunk, H), jnp.float32),
            pltpu.VMEM((H,), jnp.float32),
            pltpu.SemaphoreType.DMA,
            pltpu.SemaphoreType.DMA,
            pltpu.SemaphoreType.DMA,
            pltpu.SemaphoreType.DMA,
        ],
        compiler_params=pltpu.CompilerParams(needs_layout_passes=False),
    )
    def sc_kernel(ids_hbm, table_hbm, w_hbm, out_hbm,
                  idx_v, rows0, rows1, w_v, g0, g1, s0, s1):
        wid = lax.axis_index("s") * NC + lax.axis_index("c")
        base = wid * tw
        pltpu.sync_copy(ids_hbm.at[pl.ds(base, tw)], idx_v)
        pltpu.sync_copy(w_hbm, w_v)

        bufs = (rows0, rows1)
        gsems = (g0, g1)
        ssems = (s0, s1)

        def gather(gg, buf, gsem):
            return pltpu.make_async_copy(
                table_hbm.at[idx_v.at[pl.ds(gg * chunk, chunk)]], buf, gsem)

        def store(gg, buf, ssem):
            return pltpu.make_async_copy(
                buf, out_hbm.at[pl.ds(base + gg * chunk, chunk)], ssem)

        gather(0, bufs[0], gsems[0]).start()

        @pl.loop(0, n_chunks, step=2)
        def _pair(g):
            for b in range(2):
                gg = g + b
                nb = 1 - b
                gather(gg, bufs[b], gsems[b]).wait()

                @pl.when(gg + 1 < n_chunks)
                def _prefetch():
                    @pl.when(gg >= 1)
                    def _drain():
                        store(gg - 1, bufs[nb], ssems[nb]).wait()
                    gather(gg + 1, bufs[nb], gsems[nb]).start()

                _ln_rows(bufs[b], w_v, chunk)
                store(gg, bufs[b], ssems[b]).start()

        store(n_chunks - 2, bufs[0], ssems[0]).wait()
        store(n_chunks - 1, bufs[1], ssems[1]).wait()

    return sc_kernel


def kernel(input_ids, tok_table, ln_weight):
    b, s = input_ids.shape
    ids = input_ids.reshape(b * s)
    out = _make_sc_kernel(b * s)(ids, tok_table, ln_weight)
    return out.reshape(b, s, H)


# speedup vs baseline: 2.7029x; 1.5913x over previous
import functools

import jax
import jax.numpy as jnp
from jax import lax
from jax.experimental import pallas as pl
from jax.experimental.pallas import tpu as pltpu
from jax.experimental.pallas import tpu_sc as plsc

H = 768
L = 16
HC = H // L
NC, NS = 2, 16
NW = NC * NS
EPS = 1e-5


def _rsqrt(x):
    i = plsc.bitcast(x, jnp.int32)
    i = jnp.int32(0x5F3759DF) - lax.shift_right_arithmetic(i, 1)
    y = plsc.bitcast(i, jnp.float32)
    for _ in range(3):
        y = y * (1.5 - 0.5 * x * y * y)
    return y


def _ln_rows_2pass(rows_v, stats_v, w_v, n_rows):
    inv_h = jnp.float32(1.0 / H)

    @plsc.parallel_loop(0, n_rows)
    def _stats(r):
        s = [jnp.zeros((L,), jnp.float32) for _ in range(4)]
        q = [jnp.zeros((L,), jnp.float32) for _ in range(4)]
        for j in range(HC):
            v = rows_v[r, pl.ds(j * L, L)]
            s[j % 4] = s[j % 4] + v
            q[j % 4] = q[j % 4] + v * v
        stot = (s[0] + s[1]) + (s[2] + s[3])
        qtot = (q[0] + q[1]) + (q[2] + q[3])
        mean = _allsum(stot) * inv_h
        var = _allsum(qtot) * inv_h - mean * mean
        rstd = _rsqrt(var + EPS)
        stats_v[0, pl.ds(r * L, L)] = rstd
        stats_v[1, pl.ds(r * L, L)] = -mean * rstd

    @plsc.parallel_loop(0, n_rows)
    def _norm(r):
        rstd = stats_v[0, pl.ds(r * L, L)]
        shift = stats_v[1, pl.ds(r * L, L)]
        for j in range(HC):
            v = rows_v[r, pl.ds(j * L, L)]
            w = w_v[pl.ds(j * L, L)]
            rows_v[r, pl.ds(j * L, L)] = (v * rstd + shift) * w


def _allsum(x):
    return jnp.broadcast_to(plsc.cumsum(x)[L - 1], (L,))


def _make_sc_kernel(n_tokens):
    tw = n_tokens // NW
    chunk = 64
    n_chunks = tw // chunk

    mesh = plsc.VectorSubcoreMesh(
        core_axis_name="c", subcore_axis_name="s",
        num_cores=NC, num_subcores=NS)

    @functools.partial(
        pl.kernel,
        out_type=jax.ShapeDtypeStruct((n_tokens, H), jnp.float32),
        mesh=mesh,
        scratch_types=[
            pltpu.VMEM((tw,), jnp.int32),
            pltpu.VMEM((chunk, H), jnp.float32),
            pltpu.VMEM((chunk, H), jnp.float32),
            pltpu.VMEM((2, chunk * L), jnp.float32),
            pltpu.VMEM((2, chunk * L), jnp.float32),
            pltpu.VMEM((H,), jnp.float32),
            pltpu.SemaphoreType.DMA,
            pltpu.SemaphoreType.DMA,
            pltpu.SemaphoreType.DMA,
            pltpu.SemaphoreType.DMA,
        ],
        compiler_params=pltpu.CompilerParams(needs_layout_passes=False),
    )
    def sc_kernel(ids_hbm, table_hbm, w_hbm, out_hbm,
                  idx_v, rows0, rows1, stats0, stats1, w_v, g0, g1, s0, s1):
        wid = lax.axis_index("s") * NC + lax.axis_index("c")
        base = wid * tw
        pltpu.sync_copy(ids_hbm.at[pl.ds(base, tw)], idx_v)
        pltpu.sync_copy(w_hbm, w_v)

        bufs = (rows0, rows1)
        stats = (stats0, stats1)
        gsems = (g0, g1)
        ssems = (s0, s1)

        def gather(gg, buf, gsem):
            return pltpu.make_async_copy(
                table_hbm.at[idx_v.at[pl.ds(gg * chunk, chunk)]], buf, gsem)

        def store(gg, buf, ssem):
            return pltpu.make_async_copy(
                buf, out_hbm.at[pl.ds(base + gg * chunk, chunk)], ssem)

        gather(0, bufs[0], gsems[0]).start()

        @pl.loop(0, n_chunks, step=2)
        def _pair(g):
            for b in range(2):
                gg = g + b
                nb = 1 - b
                gather(gg, bufs[b], gsems[b]).wait()

                @pl.when(gg + 1 < n_chunks)
                def _prefetch():
                    @pl.when(gg >= 1)
                    def _drain():
                        store(gg - 1, bufs[nb], ssems[nb]).wait()
                    gather(gg + 1, bufs[nb], gsems[nb]).start()

                _ln_rows_2pass(bufs[b], stats[b], w_v, chunk)
                store(gg, bufs[b], ssems[b]).start()

        store(n_chunks - 2, bufs[0], ssems[0]).wait()
        store(n_chunks - 1, bufs[1], ssems[1]).wait()

    return sc_kernel


def kernel(input_ids, tok_table, ln_weight):
    b, s = input_ids.shape
    ids = input_ids.reshape(b * s)
    out = _make_sc_kernel(b * s)(ids, tok_table, ln_weight)
    return out.reshape(b, s, H)
